# Initial kernel scaffold; baseline (speedup 1.0000x reference)
import jax
import jax.numpy as jnp
from jax.experimental import pallas as pl


def kernel(x, edge_index, edge_attr, W_out, b_out, W_in, b_in, W_node, b_node):
    raise NotImplementedError("write your pallas kernel here")


# speedup vs baseline: 2.3199x; 2.3199x over previous
import functools

import jax
import jax.numpy as jnp
from jax import lax
from jax.experimental import pallas as pl
from jax.experimental.pallas import tpu as pltpu
from jax.experimental.pallas import tpu_sc as plsc

D = 128
DE = 16
H = 64
NEG = -1e30
WORKERS = 32
LANES = 16


@functools.lru_cache(maxsize=None)
def _build(n, e):
    nblk = n // 5 if n % 5 == 0 else n // 8
    while n % nblk or nblk % 8:
        nblk //= 2
    eblk = e // 80

    def a_body(x_ref, wi_ref, wo_ref, o_ref):
        xb = x_ref[...]
        dn = (((1,), (1,)), ((), ()))
        o_ref[0] = lax.dot_general(xb, wi_ref[...], dn,
                                   preferred_element_type=jnp.float32)
        o_ref[1] = lax.dot_general(xb, wo_ref[...], dn,
                                   preferred_element_type=jnp.float32)

    phase_a = pl.pallas_call(
        a_body,
        grid=(n // nblk,),
        in_specs=[pl.BlockSpec((nblk, D), lambda i: (i, 0)),
                  pl.BlockSpec((H, D), lambda i: (0, 0)),
                  pl.BlockSpec((H, D), lambda i: (0, 0))],
        out_specs=pl.BlockSpec((2, nblk, H), lambda i: (0, i, 0)),
        out_shape=jax.ShapeDtypeStruct((2, n, H), jnp.float32),
    )

    def b_body(ea_ref, row_ref, col_ref, wei_ref, weo_ref, bi_ref, bo_ref,
               bsel_ref, gidx_ref):
        ea = ea_ref[...]
        r = row_ref[...]
        c = col_ref[...]
        dn = (((1,), (1,)), ((), ()))
        bo = lax.dot_general(ea, weo_ref[...], dn,
                             preferred_element_type=jnp.float32) + bo_ref[...]
        bi = lax.dot_general(ea, wei_ref[...], dn,
                             preferred_element_type=jnp.float32) + bi_ref[...]
        lt = r < c
        gt = r > c
        bsel_ref[...] = jnp.where(lt, bo, jnp.where(gt, bi, NEG))
        gidx_ref[...] = r + jnp.where(lt, n, 0).astype(jnp.int32)

    phase_b = pl.pallas_call(
        b_body,
        grid=(e // eblk,),
        in_specs=[pl.BlockSpec((eblk, DE), lambda i: (i, 0)),
                  pl.BlockSpec((eblk, 1), lambda i: (i, 0)),
                  pl.BlockSpec((eblk, 1), lambda i: (i, 0)),
                  pl.BlockSpec((H, DE), lambda i: (0, 0)),
                  pl.BlockSpec((H, DE), lambda i: (0, 0)),
                  pl.BlockSpec((1, H), lambda i: (0, 0)),
                  pl.BlockSpec((1, H), lambda i: (0, 0))],
        out_specs=[pl.BlockSpec((eblk, H), lambda i: (i, 0)),
                   pl.BlockSpec((eblk, 1), lambda i: (i, 0))],
        out_shape=[jax.ShapeDtypeStruct((e, H), jnp.float32),
                   jax.ShapeDtypeStruct((e, 1), jnp.int32)],
    )

    def c_body(p1_ref, p2_ref, w1_ref, w2_ref, bn_ref, o_ref):
        fi = p1_ref[0] + p1_ref[1]
        fo = p2_ref[0] + p2_ref[1]
        dn = (((1,), (1,)), ((), ()))
        o = lax.dot_general(fi, w1_ref[...], dn,
                            preferred_element_type=jnp.float32)
        o = o + lax.dot_general(fo, w2_ref[...], dn,
                                preferred_element_type=jnp.float32)
        o_ref[...] = jnp.maximum(o + bn_ref[...], 0.0)

    nsteps = n // nblk
    phase_c = pl.pallas_call(
        c_body,
        grid=(nsteps,),
        in_specs=[pl.BlockSpec((2, nblk, H), lambda i: (0, i, 0)),
                  pl.BlockSpec((2, nblk, H), lambda i: (0, i + nsteps, 0)),
                  pl.BlockSpec((D, H), lambda i: (0, 0)),
                  pl.BlockSpec((D, H), lambda i: (0, 0)),
                  pl.BlockSpec((1, D), lambda i: (0, 0))],
        out_specs=pl.BlockSpec((nblk, D), lambda i: (i, 0)),
        out_shape=jax.ShapeDtypeStruct((n, D), jnp.float32),
    )

    return phase_a, phase_b, phase_c


@functools.lru_cache(maxsize=None)
def _build_sc(n, e):
    ew = e // WORKERS
    bk = 80
    while ew % bk:
        bk -= 8
    nbk = ew // bk
    nchunk = 50
    cs = (2 * n) // nchunk

    mesh = plsc.VectorSubcoreMesh(core_axis_name="c", subcore_axis_name="s",
                                  num_cores=2, num_subcores=16)

    @functools.partial(
        pl.kernel, mesh=mesh,
        compiler_params=pltpu.CompilerParams(use_tc_tiling_on_sc=False),
        out_type=jax.ShapeDtypeStruct((2, 2 * n, H), jnp.float32),
        scratch_types=[
            pltpu.VMEM((bk,), jnp.int32),
            pltpu.VMEM((bk, H), jnp.float32),
            pltpu.VMEM((bk, H), jnp.float32),
            pltpu.VMEM((cs, H), jnp.float32),
            pltpu.VMEM_SHARED((2 * n, H), jnp.float32),
            pltpu.SemaphoreType.DMA,
            pltpu.SemaphoreType.DMA,
        ],
    )
    def phase_sc(t_hbm, b_hbm, idx_hbm, z_hbm, out_hbm,
                 idx_v, gath_v, bsel_v, bounce_v, acc_sh, sem_g, sem_b):
        cid = lax.axis_index("c")
        sid = lax.axis_index("s")
        wid = sid * 2 + cid

        pltpu.sync_copy(z_hbm, bounce_v)
        for k in range(nchunk):
            @pl.when(sid == k % 16)
            def _():
                pltpu.sync_copy(bounce_v, acc_sh.at[pl.ds(k * cs, cs)])
        plsc.subcore_barrier()

        def block(i, carry):
            base = wid * ew + i * bk
            pltpu.sync_copy(idx_hbm.at[pl.ds(base, bk)], idx_v)
            gcp = pltpu.async_copy(t_hbm.at[idx_v], gath_v, sem_g)
            bcp = pltpu.async_copy(b_hbm.at[pl.ds(base, bk)], bsel_v, sem_b)
            gcp.wait()
            bcp.wait()

            def edge(ei, c2):
                for j in range(H // LANES):
                    sl = pl.ds(j * LANES, LANES)
                    gath_v[ei, sl] = jnp.maximum(
                        gath_v[ei, sl] + bsel_v[ei, sl], 0.0)
                return c2

            lax.fori_loop(0, bk, edge, 0, unroll=2)
            pltpu.sync_copy(gath_v, acc_sh.at[idx_v], add=True)
            return carry

        lax.fori_loop(0, nbk, block, 0)
        plsc.subcore_barrier()

        for k in range(nchunk):
            @pl.when(sid == k % 16)
            def _():
                pltpu.sync_copy(acc_sh.at[pl.ds(k * cs, cs)], bounce_v)
                pltpu.sync_copy(bounce_v, out_hbm.at[cid, pl.ds(k * cs, cs)])

    return phase_sc, cs


def kernel(x, edge_index, edge_attr, W_out, b_out, W_in, b_in, W_node,
           b_node):
    n = x.shape[0]
    e = edge_attr.shape[0]
    phase_a, phase_b, phase_c = _build(n, e)
    phase_sc, cs = _build_sc(n, e)

    row = edge_index[0].reshape(e, 1)
    col = edge_index[1].reshape(e, 1)

    t = phase_a(x, W_in[:, :D], W_out[:, :D]).reshape(2 * n, H)
    bsel, gidx = phase_b(edge_attr, row, col, W_in[:, D:], W_out[:, D:],
                         b_in.reshape(1, H), b_out.reshape(1, H))
    zeros = jnp.zeros((cs, H), jnp.float32)
    partials = phase_sc(t, bsel, gidx.reshape(e), zeros)
    return phase_c(partials, partials, W_node[:, :H], W_node[:, H:],
                   b_node.reshape(1, D))
